# baseline (speedup 1.0000x reference)
import functools

import jax
import jax.numpy as jnp
from jax import lax
from jax.experimental import pallas as pl
from jax.experimental.pallas import tpu as pltpu

B, P, O = 32, 32768, 8
NUM_CLASSES = 2
THRESHOLD = 0.5
NEGPOS_RATIO = 3
VAR0, VAR1 = 0.1, 0.2
PR, PC = 256, 128


def _smooth_l1(x):
    ax = jnp.abs(x)
    return jnp.where(ax < 1.0, 0.5 * x * x, ax - 0.5)


def _dense_kernel(loc_ref, conf_ref, has_ref, size_ref, off_ref, pri_ref,
                  s4_ref, s2_ref, tgt_ref, part_ref, key_ref, hterm_ref):
    f32 = jnp.float32
    loci = loc_ref[0]
    confi = conf_ref[0]
    hasi = has_ref[0]
    sizei = size_ref[0]
    offi = off_ref[0]
    pri = pri_ref[...]

    def deint(x, s, k):
        y = jax.lax.dot(x, s[...], precision=lax.Precision.HIGHEST)
        return [y[:, c * PC:(c + 1) * PC] for c in range(k)]

    loc = deint(loci, s4_ref, 4)
    conf = deint(confi, s2_ref, 2)
    hasd = deint(hasi, s2_ref, 2)
    sized = deint(sizei, s2_ref, 2)
    offd = deint(offi, s2_ref, 2)

    pcx, pcy, pw, ph = pri[0], pri[1], pri[2], pri[3]
    px1 = pcx - pw / 2.0
    py1 = pcy - ph / 2.0
    px2 = pcx + pw / 2.0
    py2 = pcy + ph / 2.0
    area_b = (px2 - px1) * (py2 - py1)

    iota_r = lax.broadcasted_iota(jnp.int32, (PR, PC), 0)
    iota_c = lax.broadcasted_iota(jnp.int32, (PR, PC), 1)
    iota_flat = iota_r * PC + iota_c

    ts = [[tgt_ref[0, j, c] for c in range(10)] for j in range(O)]

    bto = jnp.full((PR, PC), -1.0, f32)
    bti = jnp.zeros((PR, PC), jnp.int32)
    bp_idx = []
    for j in range(O):
        ax1, ay1, ax2, ay2 = ts[j][0], ts[j][1], ts[j][2], ts[j][3]
        area_a = (ax2 - ax1) * (ay2 - ay1)
        iw = jnp.clip(jnp.minimum(ax2, px2) - jnp.maximum(ax1, px1), 0.0, None)
        ih = jnp.clip(jnp.minimum(ay2, py2) - jnp.maximum(ay1, py1), 0.0, None)
        inter = iw * ih
        ratio = inter / (area_a + area_b - inter)
        m = jnp.max(ratio)
        bp_idx.append(jnp.min(jnp.where(ratio == m, iota_flat, jnp.int32(P))))
        upd = ratio > bto
        bto = jnp.where(upd, ratio, bto)
        bti = jnp.where(upd, j, bti)

    forced = jnp.zeros((PR, PC), jnp.bool_)
    for j in range(O):
        msk = iota_flat == bp_idx[j]
        forced = forced | msk
        bti = jnp.where(msk, j, bti)

    pos = (bto >= THRESHOLD) | forced
    posf = pos.astype(f32)

    m0 = (bti & 1) == 1
    m1 = (bti & 2) == 2
    m2 = (bti & 4) == 4

    def gather(vals):
        a0 = jnp.where(m0, vals[1], vals[0])
        a1 = jnp.where(m0, vals[3], vals[2])
        a2 = jnp.where(m0, vals[5], vals[4])
        a3 = jnp.where(m0, vals[7], vals[6])
        b0 = jnp.where(m1, a1, a0)
        b1 = jnp.where(m1, a3, a2)
        return jnp.where(m2, b1, b0)

    acx = gather([(ts[j][0] + ts[j][2]) / 2.0 for j in range(O)])
    acy = gather([(ts[j][1] + ts[j][3]) / 2.0 for j in range(O)])
    law = gather([jnp.log(ts[j][2] - ts[j][0]) for j in range(O)])
    lah = gather([jnp.log(ts[j][3] - ts[j][1]) for j in range(O)])
    hl = gather([ts[j][4] for j in range(O)])
    sz0 = gather([ts[j][5] for j in range(O)])
    sz1 = gather([ts[j][6] for j in range(O)])
    of0 = gather([ts[j][7] for j in range(O)])
    of1 = gather([ts[j][8] for j in range(O)])

    vpw = VAR0 * pw
    vph = VAR0 * ph
    lt0 = (acx - pcx) / vpw
    lt1 = (acy - pcy) / vph
    lt2 = (law - jnp.log(pw)) / VAR1
    lt3 = (lah - jnp.log(ph)) / VAR1
    loss_l = jnp.sum((_smooth_l1(loc[0] - lt0) + _smooth_l1(loc[1] - lt1) +
                      _smooth_l1(loc[2] - lt2) + _smooth_l1(loc[3] - lt3)) * posf)

    loss_sz = jnp.sum((_smooth_l1((sized[0] - sz0 / pw) * hl) +
                       _smooth_l1((sized[1] - sz1 / ph) * hl)) * posf)
    loss_of = jnp.sum((_smooth_l1((offd[0] - (of0 - pcx) / vpw) * hl) +
                       _smooth_l1((offd[1] - (of1 - pcy) / vph) * hl)) * posf)

    c0, c1 = conf[0], conf[1]
    cm = jnp.maximum(c0, c1)
    lse = cm + jnp.log(jnp.exp(c0 - cm) + jnp.exp(c1 - cm))
    gathered = jnp.where(pos, c1, c0)
    c_term = lse - gathered

    h0, h1 = hasd[0], hasd[1]
    hm = jnp.maximum(h0, h1)
    lse_h = hm + jnp.log(jnp.exp(h0 - hm) + jnp.exp(h1 - hm))
    g_h = jnp.where(hl >= 0.5, h1, h0)
    h_term = lse_h - g_h

    pos_c = jnp.sum(jnp.where(pos, c_term, 0.0))
    pos_h = jnp.sum(jnp.where(pos, h_term, 0.0))
    num_pos = jnp.sum(posf)

    key = jnp.where(pos, jnp.int32(-1),
                    lax.bitcast_convert_type(c_term, jnp.int32))
    keff = jnp.minimum(jnp.minimum(NEGPOS_RATIO * num_pos, float(P - 1)),
                       float(P) - num_pos)

    key_ref[0] = key
    hterm_ref[0] = h_term

    iota_o = lax.broadcasted_iota(jnp.int32, (PC,), 0)
    vals = [loss_l, loss_sz, loss_of, pos_c, pos_h, num_pos, keff]
    acc = jnp.zeros((PC,), f32)
    for i, v in enumerate(vals):
        acc = acc + jnp.where(iota_o == i, v, 0.0)
    part_ref[0, 0, :] = acc


def _select_kernel(key_ref, hterm_ref, part_ref, out_ref):
    f32 = jnp.float32
    key = key_ref[...]
    hterm = hterm_ref[...]
    part = part_ref[...]
    keff = part[:, :, 6:7].astype(jnp.int32)

    iota_flat = (lax.broadcasted_iota(jnp.int32, (1, PR, PC), 1) * PC +
                 lax.broadcasted_iota(jnp.int32, (1, PR, PC), 2))

    def vbody(_, lh):
        lo, hi = lh
        mid = lo + (hi - lo) // 2
        cnt = jnp.sum(jnp.where(key >= mid, 1, 0), axis=(1, 2), keepdims=True)
        take = cnt >= keff
        return jnp.where(take, mid, lo), jnp.where(take, hi, mid)

    lo0 = jnp.zeros((B, 1, 1), jnp.int32)
    hi0 = jnp.full((B, 1, 1), 2**31 - 1, jnp.int32)
    tau, _ = lax.fori_loop(0, 31, vbody, (lo0, hi0))

    gt = key > tau
    cnt_gt = jnp.sum(jnp.where(gt, 1, 0), axis=(1, 2), keepdims=True)
    tie = key == tau
    tie_need = keff - cnt_gt

    def ibody(_, lh):
        lo, hi = lh
        mid = lo + (hi - lo) // 2
        cnt = jnp.sum(jnp.where(tie & (iota_flat < mid), 1, 0),
                      axis=(1, 2), keepdims=True)
        take = cnt >= tie_need
        return jnp.where(take, lo, mid), jnp.where(take, mid, hi)

    zi = jnp.zeros((B, 1, 1), jnp.int32)
    _, cut = lax.fori_loop(0, 16, ibody, (zi, jnp.full((B, 1, 1), P, jnp.int32)))
    tie_sel = tie & (iota_flat < cut)

    lcm = lax.bitcast_convert_type(jnp.maximum(key, 0), f32)
    tau_val = jnp.where(tie_need > 0,
                        lax.bitcast_convert_type(jnp.maximum(tau, 0), f32), 0.0)
    neg_c = (jnp.sum(jnp.where(gt, lcm, 0.0)) +
             jnp.sum(tie_need.astype(f32) * tau_val))
    neg_h = jnp.sum(jnp.where(gt | tie_sel, hterm, 0.0))

    sums = jnp.sum(part[:, 0, :], axis=0)
    n = sums[5]
    vals = [sums[0] / n, (sums[3] + neg_c) / n, sums[1] / n, sums[2] / n,
            (sums[4] + neg_h) / n]
    iota_o = lax.broadcasted_iota(jnp.int32, (PC,), 0)
    acc = jnp.zeros((PC,), f32)
    for i, v in enumerate(vals):
        acc = acc + jnp.where(iota_o == i, v, 0.0)
    out_ref[0, 0, :] = acc


def kernel(loc_data, conf_data, priors, has_lp_data, size_lp_data, offset_data,
           targets):
    f32 = jnp.float32
    locT = loc_data.reshape(B, PR, 4 * PC)
    confT = conf_data.reshape(B, PR, 2 * PC)
    hasT = has_lp_data.reshape(B, PR, 2 * PC)
    sizeT = size_lp_data.reshape(B, PR, 2 * PC)
    offT = offset_data.reshape(B, PR, 2 * PC)
    priT = priors.transpose(1, 0).reshape(4, PR, PC)
    io4r = lax.broadcasted_iota(jnp.int32, (4 * PC, 4 * PC), 0)
    io4c = lax.broadcasted_iota(jnp.int32, (4 * PC, 4 * PC), 1)
    s4 = (io4r == 4 * (io4c % PC) + io4c // PC).astype(f32)
    io2r = lax.broadcasted_iota(jnp.int32, (2 * PC, 2 * PC), 0)
    io2c = lax.broadcasted_iota(jnp.int32, (2 * PC, 2 * PC), 1)
    s2 = (io2r == 2 * (io2c % PC) + io2c // PC).astype(f32)
    tgt = targets.reshape(B, O, 10)

    part, key, hterm = pl.pallas_call(
        _dense_kernel,
        grid=(B,),
        in_specs=[
            pl.BlockSpec((1, PR, 4 * PC), lambda i: (i, 0, 0)),
            pl.BlockSpec((1, PR, 2 * PC), lambda i: (i, 0, 0)),
            pl.BlockSpec((1, PR, 2 * PC), lambda i: (i, 0, 0)),
            pl.BlockSpec((1, PR, 2 * PC), lambda i: (i, 0, 0)),
            pl.BlockSpec((1, PR, 2 * PC), lambda i: (i, 0, 0)),
            pl.BlockSpec((4, PR, PC), lambda i: (0, 0, 0)),
            pl.BlockSpec((4 * PC, 4 * PC), lambda i: (0, 0)),
            pl.BlockSpec((2 * PC, 2 * PC), lambda i: (0, 0)),
            pl.BlockSpec((1, O, 10), lambda i: (i, 0, 0),
                         memory_space=pltpu.SMEM),
        ],
        out_specs=[
            pl.BlockSpec((1, 1, PC), lambda i: (i, 0, 0)),
            pl.BlockSpec((1, PR, PC), lambda i: (i, 0, 0)),
            pl.BlockSpec((1, PR, PC), lambda i: (i, 0, 0)),
        ],
        out_shape=[
            jax.ShapeDtypeStruct((B, 1, PC), f32),
            jax.ShapeDtypeStruct((B, PR, PC), jnp.int32),
            jax.ShapeDtypeStruct((B, PR, PC), f32),
        ],
    )(locT, confT, hasT, sizeT, offT, priT, s4, s2, tgt)

    out = pl.pallas_call(
        _select_kernel,
        grid=(1,),
        in_specs=[
            pl.BlockSpec((B, PR, PC), lambda i: (0, 0, 0)),
            pl.BlockSpec((B, PR, PC), lambda i: (0, 0, 0)),
            pl.BlockSpec((B, 1, PC), lambda i: (0, 0, 0)),
        ],
        out_specs=pl.BlockSpec((1, 1, PC), lambda i: (0, 0, 0)),
        out_shape=jax.ShapeDtypeStruct((1, 1, PC), f32),
    )(key, hterm, part)

    return (out[0, 0, 0], out[0, 0, 1], out[0, 0, 2], out[0, 0, 3],
            out[0, 0, 4])


# speedup vs baseline: 1.0107x; 1.0107x over previous
import functools

import jax
import jax.numpy as jnp
from jax import lax
from jax.experimental import pallas as pl
from jax.experimental.pallas import tpu as pltpu

B, P, O = 32, 32768, 8
NUM_CLASSES = 2
THRESHOLD = 0.5
NEGPOS_RATIO = 3
VAR0, VAR1 = 0.1, 0.2
PR, PC = 256, 128


def _smooth_l1(x):
    ax = jnp.abs(x)
    return jnp.where(ax < 1.0, 0.5 * x * x, ax - 0.5)


def _dense_kernel(loc_ref, conf_ref, has_ref, size_ref, off_ref, pri_ref,
                  s4_ref, s2_ref, tgt_ref, part_ref, key_ref, hterm_ref):
    f32 = jnp.float32
    loci = loc_ref[0]
    confi = conf_ref[0]
    hasi = has_ref[0]
    sizei = size_ref[0]
    offi = off_ref[0]
    pri = pri_ref[...]

    def deint(x, s, k, prec):
        y = jax.lax.dot(x, s[...], precision=prec)
        return [y[:, c * PC:(c + 1) * PC] for c in range(k)]

    hip = lax.Precision.HIGHEST
    lop = lax.Precision.DEFAULT
    loc = deint(loci, s4_ref, 4, lop)
    conf = deint(confi, s2_ref, 2, hip)
    hasd = deint(hasi, s2_ref, 2, hip)
    sized = deint(sizei, s2_ref, 2, lop)
    offd = deint(offi, s2_ref, 2, lop)

    pcx, pcy, pw, ph = pri[0], pri[1], pri[2], pri[3]
    px1 = pcx - pw / 2.0
    py1 = pcy - ph / 2.0
    px2 = pcx + pw / 2.0
    py2 = pcy + ph / 2.0
    area_b = (px2 - px1) * (py2 - py1)

    iota_r = lax.broadcasted_iota(jnp.int32, (PR, PC), 0)
    iota_c = lax.broadcasted_iota(jnp.int32, (PR, PC), 1)
    iota_flat = iota_r * PC + iota_c

    ts = [[tgt_ref[0, j, c] for c in range(10)] for j in range(O)]

    bto = jnp.full((PR, PC), -1.0, f32)
    bti = jnp.zeros((PR, PC), jnp.int32)
    bp_idx = []
    for j in range(O):
        ax1, ay1, ax2, ay2 = ts[j][0], ts[j][1], ts[j][2], ts[j][3]
        area_a = (ax2 - ax1) * (ay2 - ay1)
        iw = jnp.clip(jnp.minimum(ax2, px2) - jnp.maximum(ax1, px1), 0.0, None)
        ih = jnp.clip(jnp.minimum(ay2, py2) - jnp.maximum(ay1, py1), 0.0, None)
        inter = iw * ih
        ratio = inter / (area_a + area_b - inter)
        m = jnp.max(ratio)
        bp_idx.append(jnp.min(jnp.where(ratio == m, iota_flat, jnp.int32(P))))
        upd = ratio > bto
        bto = jnp.where(upd, ratio, bto)
        bti = jnp.where(upd, j, bti)

    forced = jnp.zeros((PR, PC), jnp.bool_)
    for j in range(O):
        msk = iota_flat == bp_idx[j]
        forced = forced | msk
        bti = jnp.where(msk, j, bti)

    pos = (bto >= THRESHOLD) | forced
    posf = pos.astype(f32)

    m0 = (bti & 1) == 1
    m1 = (bti & 2) == 2
    m2 = (bti & 4) == 4

    def gather(vals):
        a0 = jnp.where(m0, vals[1], vals[0])
        a1 = jnp.where(m0, vals[3], vals[2])
        a2 = jnp.where(m0, vals[5], vals[4])
        a3 = jnp.where(m0, vals[7], vals[6])
        b0 = jnp.where(m1, a1, a0)
        b1 = jnp.where(m1, a3, a2)
        return jnp.where(m2, b1, b0)

    acx = gather([(ts[j][0] + ts[j][2]) / 2.0 for j in range(O)])
    acy = gather([(ts[j][1] + ts[j][3]) / 2.0 for j in range(O)])
    law = gather([jnp.log(ts[j][2] - ts[j][0]) for j in range(O)])
    lah = gather([jnp.log(ts[j][3] - ts[j][1]) for j in range(O)])
    hl = gather([ts[j][4] for j in range(O)])
    sz0 = gather([ts[j][5] for j in range(O)])
    sz1 = gather([ts[j][6] for j in range(O)])
    of0 = gather([ts[j][7] for j in range(O)])
    of1 = gather([ts[j][8] for j in range(O)])

    vpw = VAR0 * pw
    vph = VAR0 * ph
    lt0 = (acx - pcx) / vpw
    lt1 = (acy - pcy) / vph
    lt2 = (law - jnp.log(pw)) / VAR1
    lt3 = (lah - jnp.log(ph)) / VAR1
    loss_l = jnp.sum((_smooth_l1(loc[0] - lt0) + _smooth_l1(loc[1] - lt1) +
                      _smooth_l1(loc[2] - lt2) + _smooth_l1(loc[3] - lt3)) * posf)

    loss_sz = jnp.sum((_smooth_l1((sized[0] - sz0 / pw) * hl) +
                       _smooth_l1((sized[1] - sz1 / ph) * hl)) * posf)
    loss_of = jnp.sum((_smooth_l1((offd[0] - (of0 - pcx) / vpw) * hl) +
                       _smooth_l1((offd[1] - (of1 - pcy) / vph) * hl)) * posf)

    c0, c1 = conf[0], conf[1]
    cm = jnp.maximum(c0, c1)
    lse = cm + jnp.log(jnp.exp(c0 - cm) + jnp.exp(c1 - cm))
    gathered = jnp.where(pos, c1, c0)
    c_term = lse - gathered

    h0, h1 = hasd[0], hasd[1]
    hm = jnp.maximum(h0, h1)
    lse_h = hm + jnp.log(jnp.exp(h0 - hm) + jnp.exp(h1 - hm))
    g_h = jnp.where(hl >= 0.5, h1, h0)
    h_term = lse_h - g_h

    pos_c = jnp.sum(jnp.where(pos, c_term, 0.0))
    pos_h = jnp.sum(jnp.where(pos, h_term, 0.0))
    num_pos = jnp.sum(posf)

    key = jnp.where(pos, jnp.int32(-1),
                    lax.bitcast_convert_type(c_term, jnp.int32))
    keff = jnp.minimum(jnp.minimum(NEGPOS_RATIO * num_pos, float(P - 1)),
                       float(P) - num_pos)

    key_ref[0] = key
    hterm_ref[0] = h_term

    iota_o = lax.broadcasted_iota(jnp.int32, (PC,), 0)
    vals = [loss_l, loss_sz, loss_of, pos_c, pos_h, num_pos, keff]
    acc = jnp.zeros((PC,), f32)
    for i, v in enumerate(vals):
        acc = acc + jnp.where(iota_o == i, v, 0.0)
    part_ref[0, 0, :] = acc


def _select_kernel(key_ref, hterm_ref, part_ref, out_ref):
    f32 = jnp.float32
    key = key_ref[...]
    hterm = hterm_ref[...]
    part = part_ref[...]
    keff = part[:, :, 6:7].astype(jnp.int32)

    iota_flat = (lax.broadcasted_iota(jnp.int32, (1, PR, PC), 1) * PC +
                 lax.broadcasted_iota(jnp.int32, (1, PR, PC), 2))

    def vbody(_, lh):
        lo, hi = lh
        mid = lo + (hi - lo) // 2
        cnt = jnp.sum(jnp.where(key >= mid, 1, 0), axis=(1, 2), keepdims=True)
        take = cnt >= keff
        return jnp.where(take, mid, lo), jnp.where(take, hi, mid)

    lo0 = jnp.zeros((B, 1, 1), jnp.int32)
    hi0 = jnp.full((B, 1, 1), 2**31 - 1, jnp.int32)
    tau, _ = lax.fori_loop(0, 31, vbody, (lo0, hi0))

    gt = key > tau
    cnt_gt = jnp.sum(jnp.where(gt, 1, 0), axis=(1, 2), keepdims=True)
    tie = key == tau
    tie_need = keff - cnt_gt

    def ibody(_, lh):
        lo, hi = lh
        mid = lo + (hi - lo) // 2
        cnt = jnp.sum(jnp.where(tie & (iota_flat < mid), 1, 0),
                      axis=(1, 2), keepdims=True)
        take = cnt >= tie_need
        return jnp.where(take, lo, mid), jnp.where(take, mid, hi)

    zi = jnp.zeros((B, 1, 1), jnp.int32)
    _, cut = lax.fori_loop(0, 16, ibody, (zi, jnp.full((B, 1, 1), P, jnp.int32)))
    tie_sel = tie & (iota_flat < cut)

    lcm = lax.bitcast_convert_type(jnp.maximum(key, 0), f32)
    tau_val = jnp.where(tie_need > 0,
                        lax.bitcast_convert_type(jnp.maximum(tau, 0), f32), 0.0)
    neg_c = (jnp.sum(jnp.where(gt, lcm, 0.0)) +
             jnp.sum(tie_need.astype(f32) * tau_val))
    neg_h = jnp.sum(jnp.where(gt | tie_sel, hterm, 0.0))

    sums = jnp.sum(part[:, 0, :], axis=0)
    n = sums[5]
    vals = [sums[0] / n, (sums[3] + neg_c) / n, sums[1] / n, sums[2] / n,
            (sums[4] + neg_h) / n]
    iota_o = lax.broadcasted_iota(jnp.int32, (PC,), 0)
    acc = jnp.zeros((PC,), f32)
    for i, v in enumerate(vals):
        acc = acc + jnp.where(iota_o == i, v, 0.0)
    out_ref[0, 0, :] = acc


def kernel(loc_data, conf_data, priors, has_lp_data, size_lp_data, offset_data,
           targets):
    f32 = jnp.float32
    locT = loc_data.reshape(B, PR, 4 * PC)
    confT = conf_data.reshape(B, PR, 2 * PC)
    hasT = has_lp_data.reshape(B, PR, 2 * PC)
    sizeT = size_lp_data.reshape(B, PR, 2 * PC)
    offT = offset_data.reshape(B, PR, 2 * PC)
    priT = priors.transpose(1, 0).reshape(4, PR, PC)
    io4r = lax.broadcasted_iota(jnp.int32, (4 * PC, 4 * PC), 0)
    io4c = lax.broadcasted_iota(jnp.int32, (4 * PC, 4 * PC), 1)
    s4 = (io4r == 4 * (io4c % PC) + io4c // PC).astype(f32)
    io2r = lax.broadcasted_iota(jnp.int32, (2 * PC, 2 * PC), 0)
    io2c = lax.broadcasted_iota(jnp.int32, (2 * PC, 2 * PC), 1)
    s2 = (io2r == 2 * (io2c % PC) + io2c // PC).astype(f32)
    tgt = targets.reshape(B, O, 10)

    part, key, hterm = pl.pallas_call(
        _dense_kernel,
        grid=(B,),
        in_specs=[
            pl.BlockSpec((1, PR, 4 * PC), lambda i: (i, 0, 0)),
            pl.BlockSpec((1, PR, 2 * PC), lambda i: (i, 0, 0)),
            pl.BlockSpec((1, PR, 2 * PC), lambda i: (i, 0, 0)),
            pl.BlockSpec((1, PR, 2 * PC), lambda i: (i, 0, 0)),
            pl.BlockSpec((1, PR, 2 * PC), lambda i: (i, 0, 0)),
            pl.BlockSpec((4, PR, PC), lambda i: (0, 0, 0)),
            pl.BlockSpec((4 * PC, 4 * PC), lambda i: (0, 0)),
            pl.BlockSpec((2 * PC, 2 * PC), lambda i: (0, 0)),
            pl.BlockSpec((1, O, 10), lambda i: (i, 0, 0),
                         memory_space=pltpu.SMEM),
        ],
        out_specs=[
            pl.BlockSpec((1, 1, PC), lambda i: (i, 0, 0)),
            pl.BlockSpec((1, PR, PC), lambda i: (i, 0, 0)),
            pl.BlockSpec((1, PR, PC), lambda i: (i, 0, 0)),
        ],
        out_shape=[
            jax.ShapeDtypeStruct((B, 1, PC), f32),
            jax.ShapeDtypeStruct((B, PR, PC), jnp.int32),
            jax.ShapeDtypeStruct((B, PR, PC), f32),
        ],
    )(locT, confT, hasT, sizeT, offT, priT, s4, s2, tgt)

    out = pl.pallas_call(
        _select_kernel,
        grid=(1,),
        in_specs=[
            pl.BlockSpec((B, PR, PC), lambda i: (0, 0, 0)),
            pl.BlockSpec((B, PR, PC), lambda i: (0, 0, 0)),
            pl.BlockSpec((B, 1, PC), lambda i: (0, 0, 0)),
        ],
        out_specs=pl.BlockSpec((1, 1, PC), lambda i: (0, 0, 0)),
        out_shape=jax.ShapeDtypeStruct((1, 1, PC), f32),
    )(key, hterm, part)

    return (out[0, 0, 0], out[0, 0, 1], out[0, 0, 2], out[0, 0, 3],
            out[0, 0, 4])


# speedup vs baseline: 1.1097x; 1.0979x over previous
import functools

import jax
import jax.numpy as jnp
from jax import lax
from jax.experimental import pallas as pl
from jax.experimental.pallas import tpu as pltpu
from jax.experimental.pallas import tpu_sc as plsc

B, P, O = 32, 32768, 8
NUM_CLASSES = 2
THRESHOLD = 0.5
NEGPOS_RATIO = 3
VAR0, VAR1 = 0.1, 0.2
PR, PC = 256, 128


def _smooth_l1(x):
    ax = jnp.abs(x)
    return jnp.where(ax < 1.0, 0.5 * x * x, ax - 0.5)


def _dense_kernel(loc_ref, conf_ref, has_ref, size_ref, off_ref, pri_ref,
                  tgt_ref, part_ref, key_ref, hterm_ref):
    f32 = jnp.float32
    loc = loc_ref[0]
    conf = conf_ref[0]
    hasd = has_ref[0]
    sized = size_ref[0]
    offd = off_ref[0]
    pri = pri_ref[...]

    pcx, pcy, pw, ph = pri[0], pri[1], pri[2], pri[3]
    px1 = pcx - pw / 2.0
    py1 = pcy - ph / 2.0
    px2 = pcx + pw / 2.0
    py2 = pcy + ph / 2.0
    area_b = (px2 - px1) * (py2 - py1)

    iota_r = lax.broadcasted_iota(jnp.int32, (PR, PC), 0)
    iota_c = lax.broadcasted_iota(jnp.int32, (PR, PC), 1)
    iota_flat = iota_r * PC + iota_c

    ts = [[tgt_ref[0, j, c] for c in range(10)] for j in range(O)]

    bto = jnp.full((PR, PC), -1.0, f32)
    bti = jnp.zeros((PR, PC), jnp.int32)
    bp_idx = []
    for j in range(O):
        ax1, ay1, ax2, ay2 = ts[j][0], ts[j][1], ts[j][2], ts[j][3]
        area_a = (ax2 - ax1) * (ay2 - ay1)
        iw = jnp.clip(jnp.minimum(ax2, px2) - jnp.maximum(ax1, px1), 0.0, None)
        ih = jnp.clip(jnp.minimum(ay2, py2) - jnp.maximum(ay1, py1), 0.0, None)
        inter = iw * ih
        ratio = inter / (area_a + area_b - inter)
        m = jnp.max(ratio)
        bp_idx.append(jnp.min(jnp.where(ratio == m, iota_flat, jnp.int32(P))))
        upd = ratio > bto
        bto = jnp.where(upd, ratio, bto)
        bti = jnp.where(upd, j, bti)

    forced = jnp.zeros((PR, PC), jnp.bool_)
    for j in range(O):
        msk = iota_flat == bp_idx[j]
        forced = forced | msk
        bti = jnp.where(msk, j, bti)

    pos = (bto >= THRESHOLD) | forced
    posf = pos.astype(f32)

    m0 = (bti & 1) == 1
    m1 = (bti & 2) == 2
    m2 = (bti & 4) == 4

    def gather(vals):
        a0 = jnp.where(m0, vals[1], vals[0])
        a1 = jnp.where(m0, vals[3], vals[2])
        a2 = jnp.where(m0, vals[5], vals[4])
        a3 = jnp.where(m0, vals[7], vals[6])
        b0 = jnp.where(m1, a1, a0)
        b1 = jnp.where(m1, a3, a2)
        return jnp.where(m2, b1, b0)

    acx = gather([(ts[j][0] + ts[j][2]) / 2.0 for j in range(O)])
    acy = gather([(ts[j][1] + ts[j][3]) / 2.0 for j in range(O)])
    law = gather([jnp.log(ts[j][2] - ts[j][0]) for j in range(O)])
    lah = gather([jnp.log(ts[j][3] - ts[j][1]) for j in range(O)])
    hl = gather([ts[j][4] for j in range(O)])
    sz0 = gather([ts[j][5] for j in range(O)])
    sz1 = gather([ts[j][6] for j in range(O)])
    of0 = gather([ts[j][7] for j in range(O)])
    of1 = gather([ts[j][8] for j in range(O)])

    vpw = VAR0 * pw
    vph = VAR0 * ph
    lt0 = (acx - pcx) / vpw
    lt1 = (acy - pcy) / vph
    lt2 = (law - jnp.log(pw)) / VAR1
    lt3 = (lah - jnp.log(ph)) / VAR1
    loss_l = jnp.sum((_smooth_l1(loc[0] - lt0) + _smooth_l1(loc[1] - lt1) +
                      _smooth_l1(loc[2] - lt2) + _smooth_l1(loc[3] - lt3)) * posf)

    loss_sz = jnp.sum((_smooth_l1((sized[0] - sz0 / pw) * hl) +
                       _smooth_l1((sized[1] - sz1 / ph) * hl)) * posf)
    loss_of = jnp.sum((_smooth_l1((offd[0] - (of0 - pcx) / vpw) * hl) +
                       _smooth_l1((offd[1] - (of1 - pcy) / vph) * hl)) * posf)

    c0, c1 = conf[0], conf[1]
    cm = jnp.maximum(c0, c1)
    lse = cm + jnp.log(jnp.exp(c0 - cm) + jnp.exp(c1 - cm))
    gathered = jnp.where(pos, c1, c0)
    c_term = lse - gathered

    h0, h1 = hasd[0], hasd[1]
    hm = jnp.maximum(h0, h1)
    lse_h = hm + jnp.log(jnp.exp(h0 - hm) + jnp.exp(h1 - hm))
    g_h = jnp.where(hl >= 0.5, h1, h0)
    h_term = lse_h - g_h

    pos_c = jnp.sum(jnp.where(pos, c_term, 0.0))
    pos_h = jnp.sum(jnp.where(pos, h_term, 0.0))
    num_pos = jnp.sum(posf)

    key = jnp.where(pos, jnp.int32(-1),
                    lax.bitcast_convert_type(c_term, jnp.int32))
    keff = jnp.minimum(jnp.minimum(NEGPOS_RATIO * num_pos, float(P - 1)),
                       float(P) - num_pos)

    key_ref[0] = key
    hterm_ref[0] = h_term

    iota_o = lax.broadcasted_iota(jnp.int32, (PC,), 0)
    vals = [loss_l, loss_sz, loss_of, pos_c, pos_h, num_pos, keff]
    acc = jnp.zeros((PC,), f32)
    for i, v in enumerate(vals):
        acc = acc + jnp.where(iota_o == i, v, 0.0)
    part_ref[0, 0, :] = acc


def _select_kernel(key_ref, hterm_ref, part_ref, out_ref):
    f32 = jnp.float32
    key = key_ref[...]
    hterm = hterm_ref[...]
    part = part_ref[...]
    keff = part[:, :, 6:7].astype(jnp.int32)

    iota_flat = (lax.broadcasted_iota(jnp.int32, (1, PR, PC), 1) * PC +
                 lax.broadcasted_iota(jnp.int32, (1, PR, PC), 2))

    def vbody(_, lh):
        lo, hi = lh
        mid = lo + (hi - lo) // 2
        cnt = jnp.sum(jnp.where(key >= mid, 1, 0), axis=(1, 2), keepdims=True)
        take = cnt >= keff
        return jnp.where(take, mid, lo), jnp.where(take, hi, mid)

    lo0 = jnp.zeros((B, 1, 1), jnp.int32)
    hi0 = jnp.full((B, 1, 1), 2**31 - 1, jnp.int32)
    tau, _ = lax.fori_loop(0, 31, vbody, (lo0, hi0))

    gt = key > tau
    cnt_gt = jnp.sum(jnp.where(gt, 1, 0), axis=(1, 2), keepdims=True)
    tie = key == tau
    tie_need = keff - cnt_gt

    def ibody(_, lh):
        lo, hi = lh
        mid = lo + (hi - lo) // 2
        cnt = jnp.sum(jnp.where(tie & (iota_flat < mid), 1, 0),
                      axis=(1, 2), keepdims=True)
        take = cnt >= tie_need
        return jnp.where(take, lo, mid), jnp.where(take, mid, hi)

    zi = jnp.zeros((B, 1, 1), jnp.int32)
    _, cut = lax.fori_loop(0, 16, ibody, (zi, jnp.full((B, 1, 1), P, jnp.int32)))
    tie_sel = tie & (iota_flat < cut)

    lcm = lax.bitcast_convert_type(jnp.maximum(key, 0), f32)
    tau_val = jnp.where(tie_need > 0,
                        lax.bitcast_convert_type(jnp.maximum(tau, 0), f32), 0.0)
    neg_c = (jnp.sum(jnp.where(gt, lcm, 0.0)) +
             jnp.sum(tie_need.astype(f32) * tau_val))
    neg_h = jnp.sum(jnp.where(gt | tie_sel, hterm, 0.0))

    sums = jnp.sum(part[:, 0, :], axis=0)
    n = sums[5]
    vals = [sums[0] / n, (sums[3] + neg_c) / n, sums[1] / n, sums[2] / n,
            (sums[4] + neg_h) / n]
    iota_o = lax.broadcasted_iota(jnp.int32, (PC,), 0)
    acc = jnp.zeros((PC,), f32)
    for i, v in enumerate(vals):
        acc = acc + jnp.where(iota_o == i, v, 0.0)
    out_ref[0, 0, :] = acc


def kernel(loc_data, conf_data, priors, has_lp_data, size_lp_data, offset_data,
           targets):
    f32 = jnp.float32
    locT = loc_data.transpose(0, 2, 1).reshape(B, 4, PR, PC)
    confT = conf_data.transpose(0, 2, 1).reshape(B, 2, PR, PC)
    hasT = has_lp_data.transpose(0, 2, 1).reshape(B, 2, PR, PC)
    sizeT = size_lp_data.transpose(0, 2, 1).reshape(B, 2, PR, PC)
    offT = offset_data.transpose(0, 2, 1).reshape(B, 2, PR, PC)
    priT = priors.transpose(1, 0).reshape(4, PR, PC)
    tgt = targets.reshape(B, O, 10)

    part, key, hterm = pl.pallas_call(
        _dense_kernel,
        grid=(B,),
        in_specs=[
            pl.BlockSpec((1, 4, PR, PC), lambda i: (i, 0, 0, 0)),
            pl.BlockSpec((1, 2, PR, PC), lambda i: (i, 0, 0, 0)),
            pl.BlockSpec((1, 2, PR, PC), lambda i: (i, 0, 0, 0)),
            pl.BlockSpec((1, 2, PR, PC), lambda i: (i, 0, 0, 0)),
            pl.BlockSpec((1, 2, PR, PC), lambda i: (i, 0, 0, 0)),
            pl.BlockSpec((4, PR, PC), lambda i: (0, 0, 0)),
            pl.BlockSpec((1, O, 10), lambda i: (i, 0, 0),
                         memory_space=pltpu.SMEM),
        ],
        out_specs=[
            pl.BlockSpec((1, 1, PC), lambda i: (i, 0, 0)),
            pl.BlockSpec((1, PR, PC), lambda i: (i, 0, 0)),
            pl.BlockSpec((1, PR, PC), lambda i: (i, 0, 0)),
        ],
        out_shape=[
            jax.ShapeDtypeStruct((B, 1, PC), f32),
            jax.ShapeDtypeStruct((B, PR, PC), jnp.int32),
            jax.ShapeDtypeStruct((B, PR, PC), f32),
        ],
    )(locT, confT, hasT, sizeT, offT, priT, tgt)

    negs = _sc_select(key.reshape(B, P), hterm.reshape(B, P),
                      part.reshape(B, PC))
    sums = jnp.sum(part[:, 0, :6], axis=0)
    neg_c = jnp.sum(negs[:, 0])
    neg_h = jnp.sum(negs[:, 1])
    n = sums[5]
    return (sums[0] / n, (sums[3] + neg_c) / n, sums[1] / n, sums[2] / n,
            (sums[4] + neg_h) / n)


NV = P // 16


def _sc_select_body(key_hbm, ht_hbm, part_hbm, out_hbm, kv, hv, pv, hist, ov):
    f32 = jnp.float32
    i32 = jnp.int32
    w = lax.axis_index("s") * 2 + lax.axis_index("c")
    pltpu.sync_copy(key_hbm.at[w], kv)
    pltpu.sync_copy(ht_hbm.at[w], hv)
    pltpu.sync_copy(part_hbm.at[w], pv)

    iota = lax.iota(i32, 16)
    zidx = jnp.zeros((16,), i32)
    gdn = lax.GatherDimensionNumbers(offset_dims=(), collapsed_slice_dims=(0,),
                                     start_index_map=(0,))

    def lane_take(x, idx):
        return lax.gather(x, idx[:, None], gdn, slice_sizes=(1,),
                          mode=lax.GatherScatterMode.PROMISE_IN_BOUNDS)

    def sufsum(v):
        r = v
        for d in (1, 2, 4, 8):
            t = lane_take(r, jnp.minimum(iota + d, 15))
            r = r + jnp.where(iota + d < 16, t, jnp.zeros_like(t))
        return r

    def presum(v):
        r = v
        for d in (1, 2, 4, 8):
            t = lane_take(r, jnp.maximum(iota - d, 0))
            r = r + jnp.where(iota - d >= 0, t, jnp.zeros_like(t))
        return r

    def bsum(v):
        return lane_take(sufsum(v), zidx)

    keff = bsum(jnp.where(iota == 6, pv[pl.ds(0, 16)].astype(i32), 0))
    ones = jnp.ones((16,), i32)

    def zero_hist(nbuckets):
        def zb(i, c):
            hist[pl.ds(i * 16, 16)] = jnp.zeros((16,), i32)
            return c
        lax.fori_loop(0, nbuckets // 16, zb, 0)

    def build_hist(shift, maskbits, prefix_shift, prefix_val):
        def hb(i, c):
            k = kv[pl.ds(i * 16, 16)]
            msk = k >= 0
            if prefix_shift is not None:
                msk = msk & ((k >> prefix_shift) == prefix_val)
            b = (k >> shift) & maskbits
            plsc.addupdate_scatter(hist, [b], ones, mask=msk)
            return c
        lax.fori_loop(0, NV, hb, 0)

    def find_bucket(nbuckets, want):
        def fb(t, carry):
            run, beta, cntgt = carry
            c = nbuckets // 16 - 1 - t
            v = hist[pl.ds(c * 16, 16)]
            d = sufsum(v)
            e = d - v
            m = ((run + e < want) & (run + d >= want)).astype(i32)
            beta = beta + bsum(m * (c * 16 + iota))
            cntgt = cntgt + bsum(m * (run + e))
            return run + lane_take(d, zidx), beta, cntgt
        z = jnp.zeros((16,), i32)
        _, beta, cntgt = lax.fori_loop(0, nbuckets // 16, fb, (z, z, z))
        return beta, cntgt

    zero_hist(2048)
    build_hist(20, 0x7FF, None, None)
    b1, g1 = find_bucket(2048, keff)

    zero_hist(1024)
    build_hist(10, 0x3FF, 20, b1)
    k2 = keff - g1
    b2, g2 = find_bucket(1024, k2)

    zero_hist(1024)
    build_hist(0, 0x3FF, 10, (b1 << 10) | b2)
    k3 = k2 - g2
    b3, g3 = find_bucket(1024, k3)

    pos_keff = keff > 0
    tau = jnp.where(pos_keff, (b1 << 20) | (b2 << 10) | b3,
                    jnp.full((16,), 0x7F000000, i32))
    tie_need = jnp.where(pos_keff, k3 - g3, 0)
    tau_val = plsc.bitcast(tau, f32)

    def fin(i, carry):
        sc, sh, tcnt = carry
        k = kv[pl.ds(i * 16, 16)]
        h = hv[pl.ds(i * 16, 16)]
        gt = k > tau
        sc = sc + jnp.where(gt, plsc.bitcast(k, f32), 0.0)
        tm = k == tau
        tmi = tm.astype(i32)
        sel = gt | (tm & (tcnt + presum(tmi) <= tie_need))
        sh = sh + jnp.where(sel, h, 0.0)
        return sc, sh, tcnt + bsum(tmi)
    zf = jnp.zeros((16,), f32)
    sc, sh, _ = lax.fori_loop(0, NV, fin, (zf, zf, jnp.zeros((16,), i32)))
    sct = bsum(sc) + tie_need.astype(f32) * jnp.where(tie_need > 0, tau_val, 0.0)
    sht = bsum(sh)

    ov[...] = jnp.where(iota == 0, sct, 0.0) + jnp.where(iota == 1, sht, 0.0)
    pltpu.sync_copy(ov, out_hbm.at[w])


def _sc_select(key2, ht2, part2):
    f32 = jnp.float32
    run = pl.kernel(
        _sc_select_body,
        out_type=jax.ShapeDtypeStruct((B, 16), f32),
        mesh=plsc.VectorSubcoreMesh(core_axis_name="c", subcore_axis_name="s"),
        compiler_params=pltpu.CompilerParams(needs_layout_passes=False),
        scratch_types=[
            pltpu.VMEM((P,), jnp.int32),
            pltpu.VMEM((P,), f32),
            pltpu.VMEM((PC,), f32),
            pltpu.VMEM((2048,), jnp.int32),
            pltpu.VMEM((16,), f32),
        ],
    )
    return run(key2, ht2, part2)


# speedup vs baseline: 1.1277x; 1.0162x over previous
import functools

import jax
import jax.numpy as jnp
from jax import lax
from jax.experimental import pallas as pl
from jax.experimental.pallas import tpu as pltpu
from jax.experimental.pallas import tpu_sc as plsc

B, P, O = 32, 32768, 8
NUM_CLASSES = 2
THRESHOLD = 0.5
NEGPOS_RATIO = 3
VAR0, VAR1 = 0.1, 0.2
PR, PC = 256, 128


def _smooth_l1(x):
    ax = jnp.abs(x)
    return jnp.where(ax < 1.0, 0.5 * x * x, ax - 0.5)


def _dense_kernel(loc_ref, conf_ref, has_ref, size_ref, off_ref, pri_ref,
                  tgt_ref, part_ref, key_ref, hterm_ref):
    f32 = jnp.float32
    loc = loc_ref[0]
    conf = conf_ref[0]
    hasd = has_ref[0]
    sized = size_ref[0]
    offd = off_ref[0]
    pri = pri_ref[...]

    pcx, pcy, pw, ph = pri[0], pri[1], pri[2], pri[3]
    px1 = pcx - pw / 2.0
    py1 = pcy - ph / 2.0
    px2 = pcx + pw / 2.0
    py2 = pcy + ph / 2.0
    area_b = (px2 - px1) * (py2 - py1)

    iota_r = lax.broadcasted_iota(jnp.int32, (PR, PC), 0)
    iota_c = lax.broadcasted_iota(jnp.int32, (PR, PC), 1)
    iota_flat = iota_r * PC + iota_c

    ts = [[tgt_ref[0, j, c] for c in range(10)] for j in range(O)]

    bto = jnp.full((PR, PC), -1.0, f32)
    bti = jnp.zeros((PR, PC), jnp.int32)
    bp_idx = []
    for j in range(O):
        ax1, ay1, ax2, ay2 = ts[j][0], ts[j][1], ts[j][2], ts[j][3]
        area_a = (ax2 - ax1) * (ay2 - ay1)
        iw = jnp.clip(jnp.minimum(ax2, px2) - jnp.maximum(ax1, px1), 0.0, None)
        ih = jnp.clip(jnp.minimum(ay2, py2) - jnp.maximum(ay1, py1), 0.0, None)
        inter = iw * ih
        ratio = inter / (area_a + area_b - inter)
        m = jnp.max(ratio)
        bp_idx.append(jnp.min(jnp.where(ratio == m, iota_flat, jnp.int32(P))))
        upd = ratio > bto
        bto = jnp.where(upd, ratio, bto)
        bti = jnp.where(upd, j, bti)

    forced = jnp.zeros((PR, PC), jnp.bool_)
    for j in range(O):
        msk = iota_flat == bp_idx[j]
        forced = forced | msk
        bti = jnp.where(msk, j, bti)

    pos = (bto >= THRESHOLD) | forced
    posf = pos.astype(f32)

    m0 = (bti & 1) == 1
    m1 = (bti & 2) == 2
    m2 = (bti & 4) == 4

    def gather(vals):
        a0 = jnp.where(m0, vals[1], vals[0])
        a1 = jnp.where(m0, vals[3], vals[2])
        a2 = jnp.where(m0, vals[5], vals[4])
        a3 = jnp.where(m0, vals[7], vals[6])
        b0 = jnp.where(m1, a1, a0)
        b1 = jnp.where(m1, a3, a2)
        return jnp.where(m2, b1, b0)

    acx = gather([(ts[j][0] + ts[j][2]) / 2.0 for j in range(O)])
    acy = gather([(ts[j][1] + ts[j][3]) / 2.0 for j in range(O)])
    law = gather([jnp.log(ts[j][2] - ts[j][0]) for j in range(O)])
    lah = gather([jnp.log(ts[j][3] - ts[j][1]) for j in range(O)])
    hl = gather([ts[j][4] for j in range(O)])
    sz0 = gather([ts[j][5] for j in range(O)])
    sz1 = gather([ts[j][6] for j in range(O)])
    of0 = gather([ts[j][7] for j in range(O)])
    of1 = gather([ts[j][8] for j in range(O)])

    vpw = VAR0 * pw
    vph = VAR0 * ph
    lt0 = (acx - pcx) / vpw
    lt1 = (acy - pcy) / vph
    lt2 = (law - jnp.log(pw)) / VAR1
    lt3 = (lah - jnp.log(ph)) / VAR1
    loss_l = jnp.sum((_smooth_l1(loc[0] - lt0) + _smooth_l1(loc[1] - lt1) +
                      _smooth_l1(loc[2] - lt2) + _smooth_l1(loc[3] - lt3)) * posf)

    loss_sz = jnp.sum((_smooth_l1((sized[0] - sz0 / pw) * hl) +
                       _smooth_l1((sized[1] - sz1 / ph) * hl)) * posf)
    loss_of = jnp.sum((_smooth_l1((offd[0] - (of0 - pcx) / vpw) * hl) +
                       _smooth_l1((offd[1] - (of1 - pcy) / vph) * hl)) * posf)

    c0, c1 = conf[0], conf[1]
    cm = jnp.maximum(c0, c1)
    lse = cm + jnp.log(jnp.exp(c0 - cm) + jnp.exp(c1 - cm))
    gathered = jnp.where(pos, c1, c0)
    c_term = lse - gathered

    h0, h1 = hasd[0], hasd[1]
    hm = jnp.maximum(h0, h1)
    lse_h = hm + jnp.log(jnp.exp(h0 - hm) + jnp.exp(h1 - hm))
    g_h = jnp.where(hl >= 0.5, h1, h0)
    h_term = lse_h - g_h

    pos_c = jnp.sum(jnp.where(pos, c_term, 0.0))
    pos_h = jnp.sum(jnp.where(pos, h_term, 0.0))
    num_pos = jnp.sum(posf)

    key = jnp.where(pos, jnp.int32(-1),
                    lax.bitcast_convert_type(c_term, jnp.int32))
    keff = jnp.minimum(jnp.minimum(NEGPOS_RATIO * num_pos, float(P - 1)),
                       float(P) - num_pos)

    key_ref[0] = key
    hterm_ref[0] = h_term

    iota_o = lax.broadcasted_iota(jnp.int32, (PC,), 0)
    vals = [loss_l, loss_sz, loss_of, pos_c, pos_h, num_pos, keff]
    acc = jnp.zeros((PC,), f32)
    for i, v in enumerate(vals):
        acc = acc + jnp.where(iota_o == i, v, 0.0)
    part_ref[0, 0, :] = acc


def _select_kernel(key_ref, hterm_ref, part_ref, out_ref):
    f32 = jnp.float32
    key = key_ref[...]
    hterm = hterm_ref[...]
    part = part_ref[...]
    keff = part[:, :, 6:7].astype(jnp.int32)

    iota_flat = (lax.broadcasted_iota(jnp.int32, (1, PR, PC), 1) * PC +
                 lax.broadcasted_iota(jnp.int32, (1, PR, PC), 2))

    def vbody(_, lh):
        lo, hi = lh
        mid = lo + (hi - lo) // 2
        cnt = jnp.sum(jnp.where(key >= mid, 1, 0), axis=(1, 2), keepdims=True)
        take = cnt >= keff
        return jnp.where(take, mid, lo), jnp.where(take, hi, mid)

    lo0 = jnp.zeros((B, 1, 1), jnp.int32)
    hi0 = jnp.full((B, 1, 1), 2**31 - 1, jnp.int32)
    tau, _ = lax.fori_loop(0, 31, vbody, (lo0, hi0))

    gt = key > tau
    cnt_gt = jnp.sum(jnp.where(gt, 1, 0), axis=(1, 2), keepdims=True)
    tie = key == tau
    tie_need = keff - cnt_gt

    def ibody(_, lh):
        lo, hi = lh
        mid = lo + (hi - lo) // 2
        cnt = jnp.sum(jnp.where(tie & (iota_flat < mid), 1, 0),
                      axis=(1, 2), keepdims=True)
        take = cnt >= tie_need
        return jnp.where(take, lo, mid), jnp.where(take, mid, hi)

    zi = jnp.zeros((B, 1, 1), jnp.int32)
    _, cut = lax.fori_loop(0, 16, ibody, (zi, jnp.full((B, 1, 1), P, jnp.int32)))
    tie_sel = tie & (iota_flat < cut)

    lcm = lax.bitcast_convert_type(jnp.maximum(key, 0), f32)
    tau_val = jnp.where(tie_need > 0,
                        lax.bitcast_convert_type(jnp.maximum(tau, 0), f32), 0.0)
    neg_c = (jnp.sum(jnp.where(gt, lcm, 0.0)) +
             jnp.sum(tie_need.astype(f32) * tau_val))
    neg_h = jnp.sum(jnp.where(gt | tie_sel, hterm, 0.0))

    sums = jnp.sum(part[:, 0, :], axis=0)
    n = sums[5]
    vals = [sums[0] / n, (sums[3] + neg_c) / n, sums[1] / n, sums[2] / n,
            (sums[4] + neg_h) / n]
    iota_o = lax.broadcasted_iota(jnp.int32, (PC,), 0)
    acc = jnp.zeros((PC,), f32)
    for i, v in enumerate(vals):
        acc = acc + jnp.where(iota_o == i, v, 0.0)
    out_ref[0, 0, :] = acc


def kernel(loc_data, conf_data, priors, has_lp_data, size_lp_data, offset_data,
           targets):
    f32 = jnp.float32
    locT = loc_data.transpose(0, 2, 1).reshape(B, 4, PR, PC)
    confT = conf_data.transpose(0, 2, 1).reshape(B, 2, PR, PC)
    hasT = has_lp_data.transpose(0, 2, 1).reshape(B, 2, PR, PC)
    sizeT = size_lp_data.transpose(0, 2, 1).reshape(B, 2, PR, PC)
    offT = offset_data.transpose(0, 2, 1).reshape(B, 2, PR, PC)
    priT = priors.transpose(1, 0).reshape(4, PR, PC)
    tgt = targets.reshape(B, O, 10)

    part, key, hterm = pl.pallas_call(
        _dense_kernel,
        grid=(B,),
        in_specs=[
            pl.BlockSpec((1, 4, PR, PC), lambda i: (i, 0, 0, 0)),
            pl.BlockSpec((1, 2, PR, PC), lambda i: (i, 0, 0, 0)),
            pl.BlockSpec((1, 2, PR, PC), lambda i: (i, 0, 0, 0)),
            pl.BlockSpec((1, 2, PR, PC), lambda i: (i, 0, 0, 0)),
            pl.BlockSpec((1, 2, PR, PC), lambda i: (i, 0, 0, 0)),
            pl.BlockSpec((4, PR, PC), lambda i: (0, 0, 0)),
            pl.BlockSpec((1, O, 10), lambda i: (i, 0, 0),
                         memory_space=pltpu.SMEM),
        ],
        out_specs=[
            pl.BlockSpec((1, 1, PC), lambda i: (i, 0, 0)),
            pl.BlockSpec((1, PR, PC), lambda i: (i, 0, 0)),
            pl.BlockSpec((1, PR, PC), lambda i: (i, 0, 0)),
        ],
        out_shape=[
            jax.ShapeDtypeStruct((B, 1, PC), f32),
            jax.ShapeDtypeStruct((B, PR, PC), jnp.int32),
            jax.ShapeDtypeStruct((B, PR, PC), f32),
        ],
    )(locT, confT, hasT, sizeT, offT, priT, tgt)

    negs = _sc_select(key.reshape(B, P), hterm.reshape(B, P),
                      part.reshape(B, PC))
    sums = jnp.sum(part[:, 0, :6], axis=0)
    neg_c = jnp.sum(negs[:, 0])
    neg_h = jnp.sum(negs[:, 1])
    n = sums[5]
    return (sums[0] / n, (sums[3] + neg_c) / n, sums[1] / n, sums[2] / n,
            (sums[4] + neg_h) / n)


NV = P // 16


def _sc_select_body(key_hbm, ht_hbm, part_hbm, out_hbm, kv, hv, pv, hist, ov):
    f32 = jnp.float32
    i32 = jnp.int32
    w = lax.axis_index("s") * 2 + lax.axis_index("c")
    pltpu.sync_copy(key_hbm.at[w], kv)
    pltpu.sync_copy(ht_hbm.at[w], hv)
    pltpu.sync_copy(part_hbm.at[w], pv)

    iota = lax.iota(i32, 16)
    zidx = jnp.zeros((16,), i32)
    gdn = lax.GatherDimensionNumbers(offset_dims=(), collapsed_slice_dims=(0,),
                                     start_index_map=(0,))

    def lane_take(x, idx):
        return lax.gather(x, idx[:, None], gdn, slice_sizes=(1,),
                          mode=lax.GatherScatterMode.PROMISE_IN_BOUNDS)

    def sufsum(v):
        r = v
        for d in (1, 2, 4, 8):
            t = lane_take(r, jnp.minimum(iota + d, 15))
            r = r + jnp.where(iota + d < 16, t, jnp.zeros_like(t))
        return r

    def presum(v):
        r = v
        for d in (1, 2, 4, 8):
            t = lane_take(r, jnp.maximum(iota - d, 0))
            r = r + jnp.where(iota - d >= 0, t, jnp.zeros_like(t))
        return r

    def bsum(v):
        return lane_take(sufsum(v), zidx)

    keff = bsum(jnp.where(iota == 6, pv[pl.ds(0, 16)].astype(i32), 0))
    ones = jnp.ones((16,), i32)

    def zero_hist(nbuckets):
        def zb(i, c):
            hist[pl.ds(i * 16, 16)] = jnp.zeros((16,), i32)
            return c
        lax.fori_loop(0, nbuckets // 16, zb, 0, unroll=8)

    def build_hist(shift, maskbits, prefix_shift, prefix_val):
        def hb(i, c):
            k = kv[pl.ds(i * 16, 16)]
            msk = k >= 0
            if prefix_shift is not None:
                msk = msk & ((k >> prefix_shift) == prefix_val)
            b = (k >> shift) & maskbits
            plsc.addupdate_scatter(hist, [b], ones, mask=msk)
            return c
        lax.fori_loop(0, NV, hb, 0, unroll=8)

    def find_bucket(nbuckets, want):
        def fb(t, carry):
            run, beta, cntgt = carry
            c = nbuckets // 16 - 1 - t
            v = hist[pl.ds(c * 16, 16)]
            d = sufsum(v)
            e = d - v
            m = ((run + e < want) & (run + d >= want)).astype(i32)
            beta = beta + bsum(m * (c * 16 + iota))
            cntgt = cntgt + bsum(m * (run + e))
            return run + lane_take(d, zidx), beta, cntgt
        z = jnp.zeros((16,), i32)
        _, beta, cntgt = lax.fori_loop(0, nbuckets // 16, fb, (z, z, z), unroll=4)
        return beta, cntgt

    zero_hist(2048)
    build_hist(20, 0x7FF, None, None)
    b1, g1 = find_bucket(2048, keff)

    zero_hist(1024)
    build_hist(10, 0x3FF, 20, b1)
    k2 = keff - g1
    b2, g2 = find_bucket(1024, k2)

    zero_hist(1024)
    build_hist(0, 0x3FF, 10, (b1 << 10) | b2)
    k3 = k2 - g2
    b3, g3 = find_bucket(1024, k3)

    pos_keff = keff > 0
    tau = jnp.where(pos_keff, (b1 << 20) | (b2 << 10) | b3,
                    jnp.full((16,), 0x7F000000, i32))
    tie_need = jnp.where(pos_keff, k3 - g3, 0)
    tau_val = plsc.bitcast(tau, f32)

    def fin(i, carry):
        sc, sh, tcnt = carry
        k = kv[pl.ds(i * 16, 16)]
        h = hv[pl.ds(i * 16, 16)]
        gt = k > tau
        sc = sc + jnp.where(gt, plsc.bitcast(k, f32), 0.0)
        tm = k == tau
        tmi = tm.astype(i32)
        sel = gt | (tm & (tcnt + presum(tmi) <= tie_need))
        sh = sh + jnp.where(sel, h, 0.0)
        return sc, sh, tcnt + bsum(tmi)
    zf = jnp.zeros((16,), f32)
    sc, sh, _ = lax.fori_loop(0, NV, fin, (zf, zf, jnp.zeros((16,), i32)), unroll=4)
    sct = bsum(sc) + tie_need.astype(f32) * jnp.where(tie_need > 0, tau_val, 0.0)
    sht = bsum(sh)

    ov[...] = jnp.where(iota == 0, sct, 0.0) + jnp.where(iota == 1, sht, 0.0)
    pltpu.sync_copy(ov, out_hbm.at[w])


def _sc_select(key2, ht2, part2):
    f32 = jnp.float32
    run = pl.kernel(
        _sc_select_body,
        out_type=jax.ShapeDtypeStruct((B, 16), f32),
        mesh=plsc.VectorSubcoreMesh(core_axis_name="c", subcore_axis_name="s"),
        compiler_params=pltpu.CompilerParams(needs_layout_passes=False),
        scratch_types=[
            pltpu.VMEM((P,), jnp.int32),
            pltpu.VMEM((P,), f32),
            pltpu.VMEM((PC,), f32),
            pltpu.VMEM((2048,), jnp.int32),
            pltpu.VMEM((16,), f32),
        ],
    )
    return run(key2, ht2, part2)


# speedup vs baseline: 1.1294x; 1.0015x over previous
import jax
import jax.numpy as jnp
from jax import lax
from jax.experimental import pallas as pl
from jax.experimental.pallas import tpu as pltpu
from jax.experimental.pallas import tpu_sc as plsc

B, P, O = 32, 32768, 8
NUM_CLASSES = 2
THRESHOLD = 0.5
NEGPOS_RATIO = 3
VAR0, VAR1 = 0.1, 0.2
PR, PC = 256, 128


def _smooth_l1(x):
    ax = jnp.abs(x)
    return jnp.where(ax < 1.0, 0.5 * x * x, ax - 0.5)


def _dense_kernel(loc_ref, conf_ref, has_ref, size_ref, off_ref, pri_ref,
                  tgt_ref, part_ref, key_ref, hterm_ref):
    f32 = jnp.float32
    loc = loc_ref[0]
    conf = conf_ref[0]
    hasd = has_ref[0]
    sized = size_ref[0]
    offd = off_ref[0]
    pri = pri_ref[...]

    pcx, pcy, pw, ph = pri[0], pri[1], pri[2], pri[3]
    px1 = pcx - pw / 2.0
    py1 = pcy - ph / 2.0
    px2 = pcx + pw / 2.0
    py2 = pcy + ph / 2.0
    area_b = (px2 - px1) * (py2 - py1)

    iota_r = lax.broadcasted_iota(jnp.int32, (PR, PC), 0)
    iota_c = lax.broadcasted_iota(jnp.int32, (PR, PC), 1)
    iota_flat = iota_r * PC + iota_c

    ts = [[tgt_ref[0, j, c] for c in range(10)] for j in range(O)]

    bto = jnp.full((PR, PC), -1.0, f32)
    bti = jnp.zeros((PR, PC), jnp.int32)
    bp_idx = []
    for j in range(O):
        ax1, ay1, ax2, ay2 = ts[j][0], ts[j][1], ts[j][2], ts[j][3]
        area_a = (ax2 - ax1) * (ay2 - ay1)
        iw = jnp.clip(jnp.minimum(ax2, px2) - jnp.maximum(ax1, px1), 0.0, None)
        ih = jnp.clip(jnp.minimum(ay2, py2) - jnp.maximum(ay1, py1), 0.0, None)
        inter = iw * ih
        ratio = inter / (area_a + area_b - inter)
        m = jnp.max(ratio)
        bp_idx.append(jnp.min(jnp.where(ratio == m, iota_flat, jnp.int32(P))))
        upd = ratio > bto
        bto = jnp.where(upd, ratio, bto)
        bti = jnp.where(upd, j, bti)

    forced = jnp.zeros((PR, PC), jnp.bool_)
    for j in range(O):
        msk = iota_flat == bp_idx[j]
        forced = forced | msk
        bti = jnp.where(msk, j, bti)

    pos = (bto >= THRESHOLD) | forced
    posf = pos.astype(f32)

    m0 = (bti & 1) == 1
    m1 = (bti & 2) == 2
    m2 = (bti & 4) == 4

    def gather(vals):
        a0 = jnp.where(m0, vals[1], vals[0])
        a1 = jnp.where(m0, vals[3], vals[2])
        a2 = jnp.where(m0, vals[5], vals[4])
        a3 = jnp.where(m0, vals[7], vals[6])
        b0 = jnp.where(m1, a1, a0)
        b1 = jnp.where(m1, a3, a2)
        return jnp.where(m2, b1, b0)

    acx = gather([(ts[j][0] + ts[j][2]) / 2.0 for j in range(O)])
    acy = gather([(ts[j][1] + ts[j][3]) / 2.0 for j in range(O)])
    law = gather([jnp.log(ts[j][2] - ts[j][0]) for j in range(O)])
    lah = gather([jnp.log(ts[j][3] - ts[j][1]) for j in range(O)])
    hl = gather([ts[j][4] for j in range(O)])
    sz0 = gather([ts[j][5] for j in range(O)])
    sz1 = gather([ts[j][6] for j in range(O)])
    of0 = gather([ts[j][7] for j in range(O)])
    of1 = gather([ts[j][8] for j in range(O)])

    vpw = VAR0 * pw
    vph = VAR0 * ph
    lt0 = (acx - pcx) / vpw
    lt1 = (acy - pcy) / vph
    lt2 = (law - jnp.log(pw)) / VAR1
    lt3 = (lah - jnp.log(ph)) / VAR1
    loss_l = jnp.sum((_smooth_l1(loc[0] - lt0) + _smooth_l1(loc[1] - lt1) +
                      _smooth_l1(loc[2] - lt2) + _smooth_l1(loc[3] - lt3)) * posf)

    loss_sz = jnp.sum((_smooth_l1((sized[0] - sz0 / pw) * hl) +
                       _smooth_l1((sized[1] - sz1 / ph) * hl)) * posf)
    loss_of = jnp.sum((_smooth_l1((offd[0] - (of0 - pcx) / vpw) * hl) +
                       _smooth_l1((offd[1] - (of1 - pcy) / vph) * hl)) * posf)

    c0, c1 = conf[0], conf[1]
    cm = jnp.maximum(c0, c1)
    lse = cm + jnp.log(jnp.exp(c0 - cm) + jnp.exp(c1 - cm))
    gathered = jnp.where(pos, c1, c0)
    c_term = lse - gathered

    h0, h1 = hasd[0], hasd[1]
    hm = jnp.maximum(h0, h1)
    lse_h = hm + jnp.log(jnp.exp(h0 - hm) + jnp.exp(h1 - hm))
    g_h = jnp.where(hl >= 0.5, h1, h0)
    h_term = lse_h - g_h

    pos_c = jnp.sum(jnp.where(pos, c_term, 0.0))
    pos_h = jnp.sum(jnp.where(pos, h_term, 0.0))
    num_pos = jnp.sum(posf)

    key = jnp.where(pos, jnp.int32(-1),
                    lax.bitcast_convert_type(c_term, jnp.int32))
    keff = jnp.minimum(jnp.minimum(NEGPOS_RATIO * num_pos, float(P - 1)),
                       float(P) - num_pos)

    key_ref[0] = key
    hterm_ref[0] = h_term

    iota_o = lax.broadcasted_iota(jnp.int32, (PC,), 0)
    vals = [loss_l, loss_sz, loss_of, pos_c, pos_h, num_pos, keff]
    acc = jnp.zeros((PC,), f32)
    for i, v in enumerate(vals):
        acc = acc + jnp.where(iota_o == i, v, 0.0)
    part_ref[0, 0, :] = acc


def _select_kernel(key_ref, hterm_ref, part_ref, out_ref):
    f32 = jnp.float32
    key = key_ref[...]
    hterm = hterm_ref[...]
    part = part_ref[...]
    keff = part[:, :, 6:7].astype(jnp.int32)

    iota_flat = (lax.broadcasted_iota(jnp.int32, (1, PR, PC), 1) * PC +
                 lax.broadcasted_iota(jnp.int32, (1, PR, PC), 2))

    def vbody(_, lh):
        lo, hi = lh
        mid = lo + (hi - lo) // 2
        cnt = jnp.sum(jnp.where(key >= mid, 1, 0), axis=(1, 2), keepdims=True)
        take = cnt >= keff
        return jnp.where(take, mid, lo), jnp.where(take, hi, mid)

    lo0 = jnp.zeros((B, 1, 1), jnp.int32)
    hi0 = jnp.full((B, 1, 1), 2**31 - 1, jnp.int32)
    tau, _ = lax.fori_loop(0, 31, vbody, (lo0, hi0))

    gt = key > tau
    cnt_gt = jnp.sum(jnp.where(gt, 1, 0), axis=(1, 2), keepdims=True)
    tie = key == tau
    tie_need = keff - cnt_gt

    def ibody(_, lh):
        lo, hi = lh
        mid = lo + (hi - lo) // 2
        cnt = jnp.sum(jnp.where(tie & (iota_flat < mid), 1, 0),
                      axis=(1, 2), keepdims=True)
        take = cnt >= tie_need
        return jnp.where(take, lo, mid), jnp.where(take, mid, hi)

    zi = jnp.zeros((B, 1, 1), jnp.int32)
    _, cut = lax.fori_loop(0, 16, ibody, (zi, jnp.full((B, 1, 1), P, jnp.int32)))
    tie_sel = tie & (iota_flat < cut)

    lcm = lax.bitcast_convert_type(jnp.maximum(key, 0), f32)
    tau_val = jnp.where(tie_need > 0,
                        lax.bitcast_convert_type(jnp.maximum(tau, 0), f32), 0.0)
    neg_c = (jnp.sum(jnp.where(gt, lcm, 0.0)) +
             jnp.sum(tie_need.astype(f32) * tau_val))
    neg_h = jnp.sum(jnp.where(gt | tie_sel, hterm, 0.0))

    sums = jnp.sum(part[:, 0, :], axis=0)
    n = sums[5]
    vals = [sums[0] / n, (sums[3] + neg_c) / n, sums[1] / n, sums[2] / n,
            (sums[4] + neg_h) / n]
    iota_o = lax.broadcasted_iota(jnp.int32, (PC,), 0)
    acc = jnp.zeros((PC,), f32)
    for i, v in enumerate(vals):
        acc = acc + jnp.where(iota_o == i, v, 0.0)
    out_ref[0, 0, :] = acc


def kernel(loc_data, conf_data, priors, has_lp_data, size_lp_data, offset_data,
           targets):
    f32 = jnp.float32
    locT = loc_data.transpose(0, 2, 1).reshape(B, 4, PR, PC)
    confT = conf_data.transpose(0, 2, 1).reshape(B, 2, PR, PC)
    hasT = has_lp_data.transpose(0, 2, 1).reshape(B, 2, PR, PC)
    sizeT = size_lp_data.transpose(0, 2, 1).reshape(B, 2, PR, PC)
    offT = offset_data.transpose(0, 2, 1).reshape(B, 2, PR, PC)
    priT = priors.transpose(1, 0).reshape(4, PR, PC)
    tgt = targets.reshape(B, O, 10)

    part, key, hterm = pl.pallas_call(
        _dense_kernel,
        grid=(B,),
        in_specs=[
            pl.BlockSpec((1, 4, PR, PC), lambda i: (i, 0, 0, 0)),
            pl.BlockSpec((1, 2, PR, PC), lambda i: (i, 0, 0, 0)),
            pl.BlockSpec((1, 2, PR, PC), lambda i: (i, 0, 0, 0)),
            pl.BlockSpec((1, 2, PR, PC), lambda i: (i, 0, 0, 0)),
            pl.BlockSpec((1, 2, PR, PC), lambda i: (i, 0, 0, 0)),
            pl.BlockSpec((4, PR, PC), lambda i: (0, 0, 0)),
            pl.BlockSpec((1, O, 10), lambda i: (i, 0, 0),
                         memory_space=pltpu.SMEM),
        ],
        out_specs=[
            pl.BlockSpec((1, 1, PC), lambda i: (i, 0, 0)),
            pl.BlockSpec((1, PR, PC), lambda i: (i, 0, 0)),
            pl.BlockSpec((1, PR, PC), lambda i: (i, 0, 0)),
        ],
        out_shape=[
            jax.ShapeDtypeStruct((B, 1, PC), f32),
            jax.ShapeDtypeStruct((B, PR, PC), jnp.int32),
            jax.ShapeDtypeStruct((B, PR, PC), f32),
        ],
    )(locT, confT, hasT, sizeT, offT, priT, tgt)

    negs = _sc_select(key.reshape(B, P), hterm.reshape(B, P),
                      part.reshape(B, PC))
    sums = jnp.sum(part[:, 0, :6], axis=0)
    neg_c = jnp.sum(negs[:, 0])
    neg_h = jnp.sum(negs[:, 1])
    n = sums[5]
    return (sums[0] / n, (sums[3] + neg_c) / n, sums[1] / n, sums[2] / n,
            (sums[4] + neg_h) / n)


NV = P // 16


def _sc_select_body(key_hbm, ht_hbm, part_hbm, out_hbm, kv, hv, pv, hist, ov):
    f32 = jnp.float32
    i32 = jnp.int32
    w = lax.axis_index("s") * 2 + lax.axis_index("c")
    pltpu.sync_copy(key_hbm.at[w], kv)
    pltpu.sync_copy(ht_hbm.at[w], hv)
    pltpu.sync_copy(part_hbm.at[w], pv)

    iota = lax.iota(i32, 16)
    zidx = jnp.zeros((16,), i32)
    gdn = lax.GatherDimensionNumbers(offset_dims=(), collapsed_slice_dims=(0,),
                                     start_index_map=(0,))

    def lane_take(x, idx):
        return lax.gather(x, idx[:, None], gdn, slice_sizes=(1,),
                          mode=lax.GatherScatterMode.PROMISE_IN_BOUNDS)

    def sufsum(v):
        r = v
        for d in (1, 2, 4, 8):
            t = lane_take(r, jnp.minimum(iota + d, 15))
            r = r + jnp.where(iota + d < 16, t, jnp.zeros_like(t))
        return r

    def presum(v):
        r = v
        for d in (1, 2, 4, 8):
            t = lane_take(r, jnp.maximum(iota - d, 0))
            r = r + jnp.where(iota - d >= 0, t, jnp.zeros_like(t))
        return r

    def bsum(v):
        return lane_take(sufsum(v), zidx)

    keff = bsum(jnp.where(iota == 6, pv[pl.ds(0, 16)].astype(i32), 0))
    ones = jnp.ones((16,), i32)

    def zero_hist(nbuckets):
        def zb(i, c):
            hist[pl.ds(i * 16, 16)] = jnp.zeros((16,), i32)
            return c
        lax.fori_loop(0, nbuckets // 16, zb, 0, unroll=8)

    def build_hist(shift, maskbits, prefix_shift, prefix_val):
        def hb(i, c):
            k = kv[pl.ds(i * 16, 16)]
            msk = k >= 0
            if prefix_shift is not None:
                msk = msk & ((k >> prefix_shift) == prefix_val)
            b = (k >> shift) & maskbits
            plsc.addupdate_scatter(hist, [b], ones, mask=msk)
            return c
        lax.fori_loop(0, NV, hb, 0, unroll=8)

    def find_bucket(nbuckets, want):
        def fb(t, carry):
            run, beta, cntgt = carry
            c = nbuckets // 16 - 1 - t
            v = hist[pl.ds(c * 16, 16)]
            d = sufsum(v)
            e = d - v
            m = ((run + e < want) & (run + d >= want)).astype(i32)
            beta = beta + bsum(m * (c * 16 + iota))
            cntgt = cntgt + bsum(m * (run + e))
            return run + lane_take(d, zidx), beta, cntgt
        z = jnp.zeros((16,), i32)
        _, beta, cntgt = lax.fori_loop(0, nbuckets // 16, fb, (z, z, z), unroll=4)
        return beta, cntgt

    zero_hist(2048)
    build_hist(20, 0x7FF, None, None)
    b1, g1 = find_bucket(2048, keff)

    zero_hist(1024)
    build_hist(10, 0x3FF, 20, b1)
    k2 = keff - g1
    b2, g2 = find_bucket(1024, k2)

    zero_hist(1024)
    build_hist(0, 0x3FF, 10, (b1 << 10) | b2)
    k3 = k2 - g2
    b3, g3 = find_bucket(1024, k3)

    pos_keff = keff > 0
    tau = jnp.where(pos_keff, (b1 << 20) | (b2 << 10) | b3,
                    jnp.full((16,), 0x7F000000, i32))
    tie_need = jnp.where(pos_keff, k3 - g3, 0)
    tau_val = plsc.bitcast(tau, f32)

    def fin(i, carry):
        sc, sh, tcnt = carry
        k = kv[pl.ds(i * 16, 16)]
        h = hv[pl.ds(i * 16, 16)]
        gt = k > tau
        sc = sc + jnp.where(gt, plsc.bitcast(k, f32), 0.0)
        tm = k == tau
        tmi = tm.astype(i32)
        sel = gt | (tm & (tcnt + presum(tmi) <= tie_need))
        sh = sh + jnp.where(sel, h, 0.0)
        return sc, sh, tcnt + bsum(tmi)
    zf = jnp.zeros((16,), f32)
    sc, sh, _ = lax.fori_loop(0, NV, fin, (zf, zf, jnp.zeros((16,), i32)), unroll=4)
    sct = bsum(sc) + tie_need.astype(f32) * jnp.where(tie_need > 0, tau_val, 0.0)
    sht = bsum(sh)

    ov[...] = jnp.where(iota == 0, sct, 0.0) + jnp.where(iota == 1, sht, 0.0)
    pltpu.sync_copy(ov, out_hbm.at[w])


def _sc_select(key2, ht2, part2):
    f32 = jnp.float32
    run = pl.kernel(
        _sc_select_body,
        out_type=jax.ShapeDtypeStruct((B, 16), f32),
        mesh=plsc.VectorSubcoreMesh(core_axis_name="c", subcore_axis_name="s"),
        compiler_params=pltpu.CompilerParams(needs_layout_passes=False),
        scratch_types=[
            pltpu.VMEM((P,), jnp.int32),
            pltpu.VMEM((P,), f32),
            pltpu.VMEM((PC,), f32),
            pltpu.VMEM((2048,), jnp.int32),
            pltpu.VMEM((16,), f32),
        ],
    )
    return run(key2, ht2, part2)
